# baseline (speedup 1.0000x reference)
import functools

import jax
import jax.numpy as jnp
from jax import lax
from jax.experimental import pallas as pl
from jax.experimental.pallas import tpu as pltpu
from jax.experimental.pallas import tpu_sc as plsc

_B = 16384
_D = 16
_NC = 2
_NS = 16
_NW = _NC * _NS
_BPW = _B // _NW

_mesh = plsc.VectorSubcoreMesh(core_axis_name="c", subcore_axis_name="s")


@functools.partial(
    pl.kernel,
    mesh=_mesh,
    out_type=jax.ShapeDtypeStruct((_B, _D), jnp.float32),
    compiler_params=pltpu.CompilerParams(use_tc_tiling_on_sc=False),
    scratch_types=[
        pltpu.VMEM((_BPW,), jnp.int32),
        pltpu.VMEM((_BPW, _D), jnp.float32),
        pltpu.SemaphoreType.DMA,
    ],
)
def _sc_gather(index_hbm, emb_hbm, out_hbm, idx_v, rows_v, sem):
    wid = lax.axis_index("s") * _NC + lax.axis_index("c")
    base = wid * _BPW
    pltpu.sync_copy(index_hbm.at[pl.ds(base, _BPW)], idx_v)
    pltpu.async_copy(emb_hbm.at[idx_v], rows_v, sem).wait()
    pltpu.sync_copy(rows_v, out_hbm.at[pl.ds(base, _BPW)])


def kernel(data, index, emb):
    del data
    return _sc_gather(index, emb)


# speedup vs baseline: 5.6249x; 5.6249x over previous
import functools

import jax
import jax.numpy as jnp
from jax import lax
from jax.experimental import pallas as pl
from jax.experimental.pallas import tpu as pltpu
from jax.experimental.pallas import tpu_sc as plsc

_B = 16384
_D = 16
_N = 1000000
_NC = 2
_NS = 16
_NW = _NC * _NS
_BPW = _B // _NW
_W = 128
_K = 16
_TAIL = _N - (_N % _W)
_LASTBLK = (_TAIL // _W - 1) * _W
_NBATCH = _BPW // _K
_PAIRS = _NBATCH // 2

_mesh = plsc.VectorSubcoreMesh(core_axis_name="c", subcore_axis_name="s")


@functools.partial(
    pl.kernel,
    mesh=_mesh,
    out_type=jax.ShapeDtypeStruct((_D, _B), jnp.float32),
    compiler_params=pltpu.CompilerParams(
        use_tc_tiling_on_sc=True, needs_layout_passes=False
    ),
    scratch_types=[
        pltpu.VMEM((_BPW,), jnp.int32),
        pltpu.VMEM((_K * _D, _W), jnp.float32),
        pltpu.VMEM((_K * _D, _W), jnp.float32),
        pltpu.VMEM((_D, _N - _TAIL), jnp.float32),
        pltpu.VMEM((_D, _BPW), jnp.float32),
        pltpu.SemaphoreType.DMA,
        pltpu.SemaphoreType.DMA,
    ],
)
def _sc_gather(index_hbm, embt_hbm, tail_hbm, out_hbm,
               idx_v, buf0, buf1, tail_v, out_stage, sem0, sem1):
    wid = lax.axis_index("s") * _NC + lax.axis_index("c")
    base = wid * _BPW
    pltpu.sync_copy(index_hbm.at[pl.ds(base, _BPW)], idx_v)
    pltpu.sync_copy(tail_hbm, tail_v)
    rows = lax.broadcasted_iota(jnp.int32, (_D,), 0)

    def fire(b, buf, sem):
        vec = idx_v[pl.ds(b * _K, _K)]
        for k in range(_K):
            r = vec[k]
            ls = pl.multiple_of(jnp.minimum(r >> 7, _LASTBLK >> 7) << 7, _W)
            pltpu.async_copy(
                embt_hbm.at[:, pl.ds(ls, _W)],
                buf.at[pl.ds(k * _D, _D), :],
                sem,
            )

    def drain(buf, sem):
        for k in range(_K):
            pltpu.make_async_copy(
                embt_hbm.at[:, pl.ds(0, _W)],
                buf.at[pl.ds(k * _D, _D), :],
                sem,
            ).wait()

    def extract(b, buf):
        vec = idx_v[pl.ds(b * _K, _K)]
        for k in range(_K):
            i = b * _K + k
            r = vec[k]
            ls = jnp.minimum(r >> 7, _LASTBLK >> 7) << 7
            l = jnp.full((_D,), jnp.minimum(r - ls, _W - 1), jnp.int32)
            v_main = plsc.load_gather(buf, [k * _D + rows, l])
            lt = jnp.full((_D,), jnp.clip(r - _TAIL, 0, _N - _TAIL - 1),
                          jnp.int32)
            v_tail = plsc.load_gather(tail_v, [rows, lt])
            v = jnp.where(jnp.full((_D,), r >= _TAIL, jnp.bool_),
                          v_tail, v_main)
            plsc.store_scatter(
                out_stage,
                [rows, jnp.full((_D,), i, jnp.int32)],
                v,
            )

    fire(0, buf0, sem0)

    def body(t, carry):
        b0 = 2 * t
        fire(b0 + 1, buf1, sem1)
        drain(buf0, sem0)
        extract(b0, buf0)
        fire(jnp.minimum(b0 + 2, _NBATCH - 2), buf0, sem0)
        drain(buf1, sem1)
        extract(b0 + 1, buf1)
        return carry

    lax.fori_loop(0, _PAIRS, body, 0)
    drain(buf0, sem0)

    pltpu.sync_copy(out_stage, out_hbm.at[:, pl.ds(base, _BPW)])


def kernel(data, index, emb):
    del data
    embt = emb.T
    out_t = _sc_gather(index, embt, embt[:, _TAIL:])
    return out_t.T


# speedup vs baseline: 5.6764x; 1.0092x over previous
import functools

import jax
import jax.numpy as jnp
from jax import lax
from jax.experimental import pallas as pl
from jax.experimental.pallas import tpu as pltpu
from jax.experimental.pallas import tpu_sc as plsc

_B = 16384
_D = 16
_N = 1000000
_NC = 2
_NS = 16
_NW = _NC * _NS
_BPW = _B // _NW
_W = 128
_K = 16
_TAIL = _N - (_N % _W)
_LASTBLK = (_TAIL // _W - 1) * _W
_NBATCH = _BPW // _K
_PAIRS = _NBATCH // 2

_mesh = plsc.VectorSubcoreMesh(core_axis_name="c", subcore_axis_name="s")


@functools.partial(
    pl.kernel,
    mesh=_mesh,
    out_type=jax.ShapeDtypeStruct((_D, _B), jnp.float32),
    compiler_params=pltpu.CompilerParams(
        use_tc_tiling_on_sc=True, needs_layout_passes=False
    ),
    scratch_types=[
        pltpu.VMEM((_BPW,), jnp.int32),
        pltpu.VMEM((_K * _D, _W), jnp.float32),
        pltpu.VMEM((_K * _D, _W), jnp.float32),
        pltpu.VMEM((_D, _N - _TAIL), jnp.float32),
        pltpu.VMEM((_D, _BPW), jnp.float32),
        pltpu.SemaphoreType.DMA,
        pltpu.SemaphoreType.DMA,
    ],
)
def _sc_gather(index_hbm, embt_hbm, tail_hbm, out_hbm,
               idx_v, buf0, buf1, tail_v, out_stage, sem0, sem1):
    wid = lax.axis_index("s") * _NC + lax.axis_index("c")
    base = wid * _BPW
    pltpu.sync_copy(index_hbm.at[pl.ds(base, _BPW)], idx_v)
    pltpu.sync_copy(tail_hbm, tail_v)
    rows = lax.broadcasted_iota(jnp.int32, (_D,), 0)

    def fire(b, buf, sem):
        lsvec = jnp.minimum(idx_v[pl.ds(b * _K, _K)] >> 7, _LASTBLK >> 7) << 7
        for k in range(_K):
            ls = pl.multiple_of(lsvec[k], _W)
            pltpu.async_copy(
                embt_hbm.at[:, pl.ds(ls, _W)],
                buf.at[pl.ds(k * _D, _D), :],
                sem,
            )

    def drain(buf, sem):
        for k in range(_K):
            pltpu.make_async_copy(
                embt_hbm.at[:, pl.ds(0, _W)],
                buf.at[pl.ds(k * _D, _D), :],
                sem,
            ).wait()

    def extract(b, buf):
        vec = idx_v[pl.ds(b * _K, _K)]
        lvec = vec & (_W - 1)
        for k in range(_K):
            i = b * _K + k
            r = vec[k]
            col = jnp.full((_D,), i, jnp.int32)

            @pl.when(r < _TAIL)
            def _main():
                v = plsc.load_gather(
                    buf, [k * _D + rows, jnp.full((_D,), lvec[k], jnp.int32)]
                )
                plsc.store_scatter(out_stage, [rows, col], v)

            @pl.when(r >= _TAIL)
            def _tail():
                v = plsc.load_gather(
                    tail_v, [rows, jnp.full((_D,), r - _TAIL, jnp.int32)]
                )
                plsc.store_scatter(out_stage, [rows, col], v)

    fire(0, buf0, sem0)

    def body(t, carry):
        b0 = 2 * t
        fire(b0 + 1, buf1, sem1)
        drain(buf0, sem0)
        extract(b0, buf0)
        fire(jnp.minimum(b0 + 2, _NBATCH - 2), buf0, sem0)
        drain(buf1, sem1)
        extract(b0 + 1, buf1)
        return carry

    lax.fori_loop(0, _PAIRS, body, 0)
    drain(buf0, sem0)

    pltpu.sync_copy(out_stage, out_hbm.at[:, pl.ds(base, _BPW)])


def kernel(data, index, emb):
    del data
    embt = emb.T
    out_t = _sc_gather(index, embt, embt[:, _TAIL:])
    return out_t.T


# speedup vs baseline: 5.9808x; 1.0536x over previous
import functools

import jax
import jax.numpy as jnp
from jax import lax
from jax.experimental import pallas as pl
from jax.experimental.pallas import tpu as pltpu
from jax.experimental.pallas import tpu_sc as plsc

_B = 16384
_D = 16
_N = 1000000
_NC = 2
_NS = 16
_NW = _NC * _NS
_BPW = _B // _NW
_W = 128
_K = 16
_TAIL = _N - (_N % _W)
_LASTBLK = (_TAIL // _W - 1) * _W
_NBATCH = _BPW // _K
_PAIRS = _NBATCH // 2

_mesh = plsc.VectorSubcoreMesh(core_axis_name="c", subcore_axis_name="s")


@functools.partial(
    pl.kernel,
    mesh=_mesh,
    out_type=jax.ShapeDtypeStruct((_D, _B), jnp.float32),
    compiler_params=pltpu.CompilerParams(
        use_tc_tiling_on_sc=True, needs_layout_passes=False
    ),
    scratch_types=[
        pltpu.VMEM((_BPW,), jnp.int32),
        pltpu.VMEM((_K * _D, _W), jnp.float32),
        pltpu.VMEM((_K * _D, _W), jnp.float32),
        pltpu.VMEM((_K * _D, _W), jnp.float32),
        pltpu.VMEM((_D, _N - _TAIL), jnp.float32),
        pltpu.VMEM((_D, _BPW), jnp.float32),
        pltpu.SemaphoreType.DMA,
        pltpu.SemaphoreType.DMA,
        pltpu.SemaphoreType.DMA,
    ],
)
def _sc_gather(index_hbm, embt_hbm, tail_hbm, out_hbm,
               idx_v, buf0, buf1, buf2, tail_v, out_stage,
               sem0, sem1, sem2):
    wid = lax.axis_index("s") * _NC + lax.axis_index("c")
    base = wid * _BPW
    pltpu.sync_copy(index_hbm.at[pl.ds(base, _BPW)], idx_v)
    pltpu.sync_copy(tail_hbm, tail_v)
    rows = lax.broadcasted_iota(jnp.int32, (_D,), 0)

    def fire(b, buf, sem):
        lsvec = jnp.minimum(idx_v[pl.ds(b * _K, _K)] >> 7, _LASTBLK >> 7) << 7
        for k in range(_K):
            ls = pl.multiple_of(lsvec[k], _W)
            pltpu.async_copy(
                embt_hbm.at[:, pl.ds(ls, _W)],
                buf.at[pl.ds(k * _D, _D), :],
                sem,
            )

    def drain(buf, sem):
        for k in range(_K):
            pltpu.make_async_copy(
                embt_hbm.at[:, pl.ds(0, _W)],
                buf.at[pl.ds(k * _D, _D), :],
                sem,
            ).wait()

    def extract(b, buf):
        vec = idx_v[pl.ds(b * _K, _K)]
        lvec = vec & (_W - 1)
        for k in range(_K):
            i = b * _K + k
            r = vec[k]
            col = jnp.full((_D,), i, jnp.int32)

            @pl.when(r < _TAIL)
            def _main():
                v = plsc.load_gather(
                    buf, [k * _D + rows, jnp.full((_D,), lvec[k], jnp.int32)]
                )
                plsc.store_scatter(out_stage, [rows, col], v)

            @pl.when(r >= _TAIL)
            def _tail():
                v = plsc.load_gather(
                    tail_v, [rows, jnp.full((_D,), r - _TAIL, jnp.int32)]
                )
                plsc.store_scatter(out_stage, [rows, col], v)

    fire(0, buf0, sem0)
    fire(1, buf1, sem1)
    fire(2, buf2, sem2)

    def body(t, carry):
        b = 3 * t
        drain(buf0, sem0)
        extract(b, buf0)
        fire(jnp.minimum(b + 3, _NBATCH - 1), buf0, sem0)
        drain(buf1, sem1)
        extract(b + 1, buf1)
        fire(jnp.minimum(b + 4, _NBATCH - 1), buf1, sem1)
        drain(buf2, sem2)
        extract(b + 2, buf2)
        fire(jnp.minimum(b + 5, _NBATCH - 1), buf2, sem2)
        return carry

    lax.fori_loop(0, _NBATCH // 3, body, 0)
    drain(buf0, sem0)
    extract(_NBATCH - 2, buf0)
    drain(buf1, sem1)
    extract(_NBATCH - 1, buf1)
    drain(buf2, sem2)

    pltpu.sync_copy(out_stage, out_hbm.at[:, pl.ds(base, _BPW)])


def kernel(data, index, emb):
    del data
    embt = emb.T
    out_t = _sc_gather(index, embt, embt[:, _TAIL:])
    return out_t.T
